# baseline (speedup 1.0000x reference)
import functools

import jax
import jax.numpy as jnp
from jax import lax
from jax.experimental import pallas as pl
from jax.experimental.pallas import tpu as pltpu
from jax.experimental.pallas import tpu_sc as plsc

BATCH = 16384
WIDTH = 64
ROWS = 1000000

_info = plsc.get_sparse_core_info()
_NC, _NS, _L = _info.num_cores, _info.num_subcores, _info.num_lanes
_NW = _NC * _NS
_B_PER_W = BATCH // _NW
_UNROLL = 8


def _make_gather():
  mesh = plsc.VectorSubcoreMesh(core_axis_name="c", subcore_axis_name="s")

  @functools.partial(
      pl.kernel,
      mesh=mesh,
      out_type=jax.ShapeDtypeStruct((BATCH, WIDTH), jnp.float32),
      scratch_types=[
          pltpu.VMEM((_B_PER_W,), jnp.int32),
          pltpu.SemaphoreType.DMA,
      ],
  )
  def gather_kernel(y_hbm, table_hbm, out_hbm, idx_v, sem):
    wid = lax.axis_index("s") * _NC + lax.axis_index("c")
    base = wid * _B_PER_W
    pltpu.sync_copy(y_hbm.at[pl.ds(base, _B_PER_W)], idx_v)

    for j in range(0, _B_PER_W, _L):
      v = idx_v[pl.ds(j, _L)]
      for u in range(_L):
        row = v[u]
        pltpu.async_copy(
            table_hbm.at[pl.ds(row, 1)],
            out_hbm.at[pl.ds(base + j + u, 1)],
            sem,
        )

    pltpu.make_async_copy(
        table_hbm.at[pl.ds(0, _B_PER_W)],
        out_hbm.at[pl.ds(base, _B_PER_W)],
        sem,
    ).wait()

  return gather_kernel


_gather = _make_gather()


@jax.jit
def kernel(y, genre_emb):
  out = _gather(y.astype(jnp.int32), genre_emb)
  return out.reshape(BATCH, 1, WIDTH)


# speedup vs baseline: 1.6641x; 1.6641x over previous
import functools

import jax
import jax.numpy as jnp
from jax import lax
from jax.experimental import pallas as pl
from jax.experimental.pallas import tpu as pltpu
from jax.experimental.pallas import tpu_sc as plsc

BATCH = 16384
WIDTH = 64
ROWS = 1000000

_info = plsc.get_sparse_core_info()
_NC, _NS, _L = _info.num_cores, _info.num_subcores, _info.num_lanes
_NW = _NC * _NS
_B_PER_W = BATCH // _NW


def _make_gather():
  mesh = plsc.VectorSubcoreMesh(core_axis_name="c", subcore_axis_name="s")

  @functools.partial(
      pl.kernel,
      mesh=mesh,
      out_type=jax.ShapeDtypeStruct((BATCH, WIDTH), jnp.float32),
      scratch_types=[
          pltpu.VMEM((_B_PER_W,), jnp.int32),
          pltpu.VMEM((_B_PER_W, WIDTH), jnp.float32),
          pltpu.SemaphoreType.DMA,
      ],
  )
  def gather_kernel(y_hbm, table_hbm, out_hbm, idx_v, rows_v, sem):
    wid = lax.axis_index("s") * _NC + lax.axis_index("c")
    base = wid * _B_PER_W
    pltpu.sync_copy(y_hbm.at[pl.ds(base, _B_PER_W)], idx_v)

    for j in range(0, _B_PER_W, _L):
      v = idx_v[pl.ds(j, _L)]
      for u in range(_L):
        row = v[u]
        pltpu.async_copy(
            table_hbm.at[pl.ds(row, 1)],
            rows_v.at[pl.ds(j + u, 1)],
            sem,
        )

    pltpu.make_async_copy(
        table_hbm.at[pl.ds(0, _B_PER_W)],
        rows_v,
        sem,
    ).wait()

    pltpu.sync_copy(rows_v, out_hbm.at[pl.ds(base, _B_PER_W)])

  return gather_kernel


_gather = _make_gather()


@jax.jit
def kernel(y, genre_emb):
  out = _gather(y.astype(jnp.int32), genre_emb)
  return out.reshape(BATCH, 1, WIDTH)
